# baseline (speedup 1.0000x reference)
import jax
import jax.numpy as jnp
from jax import lax
from jax.experimental import pallas as pl
from jax.experimental.pallas import tpu as pltpu
from jax.experimental.pallas import tpu_sc as plsc

_BINS = 32
_EMBED = 64
_NUM_EMB = 2 * _BINS + 2
_B = 64
_L = 4096
_NC = 2
_NS = 16
_NW = _NC * _NS
_ROWS_PER_W = _B // _NW
_TOK_PER_W = _ROWS_PER_W * _L
_CHUNK = 256
_NCHUNK = _TOK_PER_W // _CHUNK
_LANES = 16


def _body(query_hbm, key_hbm, table_hbm, out_hbm, query_v, table_v, keys_v,
          idx_v, rows_v, sem_o):
    wid = lax.axis_index("s") * _NC + lax.axis_index("c")
    t0 = wid * _TOK_PER_W
    pltpu.sync_copy(query_hbm, query_v)
    pltpu.sync_copy(table_hbm, table_v)
    kr0 = pl.multiple_of(wid * (_TOK_PER_W // 128), 8)
    pltpu.sync_copy(key_hbm.at[pl.ds(kr0, _TOK_PER_W // 128)], keys_v)

    base = wid * _ROWS_PER_W
    vbase = (base // _LANES) * _LANES
    qvec = query_v[pl.ds(vbase, _LANES)]
    for r in range(_ROWS_PER_W):
        lane = base + r - vbase
        q = qvec.at[jnp.full((_LANES,), lane, jnp.int32)].get(
            mode="promise_in_bounds")

        @plsc.parallel_loop(0, _L // _LANES, unroll=4)
        def _vec(i, q=q, off=r * _L):
            t = off + i * _LANES
            kv = keys_v[t // 128, pl.ds(t % 128, _LANES)]
            d = jnp.clip(kv - q, -_BINS, _BINS) + (_BINS + 1)
            idx_v[pl.ds(t, _LANES)] = d

    col = lax.iota(jnp.int32, 16)

    def outer(oc, _):
        for p in range(2):
            c = oc * 2 + p

            @pl.when(oc > 0)
            def _wait_prev(p=p):
                pltpu.make_async_copy(
                    rows_v.at[p],
                    out_hbm.at[0, :, pl.ds(0, _CHUNK)], sem_o).wait()

            @plsc.parallel_loop(0, _CHUNK // _LANES, unroll=2)
            def _group(g, c=c, p=p):
                r_vec = idx_v[pl.ds(c * _CHUNK + g * _LANES, _LANES)]
                fla = r_vec * _EMBED
                for e in range(_EMBED):
                    v = plsc.load_gather(table_v, [fla + e])
                    rows_v[p, e, pl.ds(g * _LANES, _LANES)] = v

            tok = t0 + c * _CHUNK
            bb = tok // _L
            l0 = pl.multiple_of(tok % _L, 256)
            pltpu.async_copy(
                rows_v.at[p], out_hbm.at[bb, :, pl.ds(l0, _CHUNK)], sem_o)
        return 0

    lax.fori_loop(0, _NCHUNK // 2, outer, 0)
    for p in range(2):
        pltpu.make_async_copy(
            rows_v.at[p],
            out_hbm.at[0, :, pl.ds(0, _CHUNK)], sem_o).wait()


@jax.jit
def kernel(query_residue_index, key_residue_index, weight):
    mesh = plsc.VectorSubcoreMesh(core_axis_name="c", subcore_axis_name="s")
    run = pl.kernel(
        _body,
        out_type=jax.ShapeDtypeStruct((_B, _EMBED, _L), jnp.float32),
        mesh=mesh,
        compiler_params=pltpu.CompilerParams(needs_layout_passes=False),
        scratch_types=[
            pltpu.VMEM((_B,), jnp.int32),
            pltpu.VMEM((_NUM_EMB * _EMBED,), jnp.float32),
            pltpu.VMEM((_TOK_PER_W // 128, 128), jnp.int32),
            pltpu.VMEM((_TOK_PER_W,), jnp.int32),
            pltpu.VMEM((2, _EMBED, _CHUNK), jnp.float32),
            pltpu.SemaphoreType.DMA,
        ],
    )
    out = run(query_residue_index,
              key_residue_index.reshape(_B * _L // 128, 128),
              weight.reshape(-1))
    return out.transpose(0, 2, 1)


# speedup vs baseline: 1.7739x; 1.7739x over previous
import jax
import jax.numpy as jnp
from jax import lax
from jax.experimental import pallas as pl
from jax.experimental.pallas import tpu as pltpu
from jax.experimental.pallas import tpu_sc as plsc

_BINS = 32
_EMBED = 64
_NUM_EMB = 2 * _BINS + 2
_B = 64
_L = 4096
_NC = 2
_NS = 16
_NW = _NC * _NS
_ROWS_PER_W = _B // _NW
_TOK_PER_W = _ROWS_PER_W * _L
_CHUNK = 256
_NCHUNK = _TOK_PER_W // _CHUNK
_LANES = 16


def _body(query_hbm, key_hbm, table_hbm, out_hbm, query_v, table_v, keys_v,
          idx_v, rows_v, sem_o):
    wid = lax.axis_index("s") * _NC + lax.axis_index("c")
    t0 = wid * _TOK_PER_W
    pltpu.sync_copy(query_hbm, query_v)
    pltpu.sync_copy(table_hbm, table_v)
    kr0 = pl.multiple_of(wid * (_TOK_PER_W // 128), 8)
    pltpu.sync_copy(key_hbm.at[pl.ds(kr0, _TOK_PER_W // 128)], keys_v)

    base = wid * _ROWS_PER_W
    vbase = (base // _LANES) * _LANES
    qvec = query_v[pl.ds(vbase, _LANES)]
    for r in range(_ROWS_PER_W):
        lane = base + r - vbase
        q = qvec.at[jnp.full((_LANES,), lane, jnp.int32)].get(
            mode="promise_in_bounds")

        @plsc.parallel_loop(0, _L // _LANES, unroll=4)
        def _vec(i, q=q, off=r * _L):
            t = off + i * _LANES
            kv = keys_v[t // 128, pl.ds(t % 128, _LANES)]
            d = jnp.clip(kv - q, -_BINS, _BINS) + (_BINS + 1)
            idx_v[pl.ds(t, _LANES)] = d

    col67 = lax.iota(jnp.int32, 16) * 67

    def outer(oc, _):
        for p in range(2):
            c = oc * 2 + p

            @pl.when(oc > 0)
            def _wait_prev(p=p):
                pltpu.make_async_copy(
                    rows_v.at[p],
                    out_hbm.at[0, :, pl.ds(0, _CHUNK)], sem_o).wait()

            @plsc.parallel_loop(0, _CHUNK // _LANES, unroll=2)
            def _group(g, c=c, p=p):
                r_vec = idx_v[pl.ds(c * _CHUNK + g * _LANES, _LANES)]
                addr0 = r_vec + col67
                for e in range(_EMBED):
                    v = plsc.load_gather(table_v, [addr0 + e * (16 * 67)])
                    rows_v[p, e, pl.ds(g * _LANES, _LANES)] = v

            tok = t0 + c * _CHUNK
            bb = tok // _L
            l0 = pl.multiple_of(tok % _L, 256)
            pltpu.async_copy(
                rows_v.at[p], out_hbm.at[bb, :, pl.ds(l0, _CHUNK)], sem_o)
        return 0

    lax.fori_loop(0, _NCHUNK // 2, outer, 0)
    for p in range(2):
        pltpu.make_async_copy(
            rows_v.at[p],
            out_hbm.at[0, :, pl.ds(0, _CHUNK)], sem_o).wait()


@jax.jit
def kernel(query_residue_index, key_residue_index, weight):
    mesh = plsc.VectorSubcoreMesh(core_axis_name="c", subcore_axis_name="s")
    run = pl.kernel(
        _body,
        out_type=jax.ShapeDtypeStruct((_B, _EMBED, _L), jnp.float32),
        mesh=mesh,
        compiler_params=pltpu.CompilerParams(needs_layout_passes=False),
        scratch_types=[
            pltpu.VMEM((_B,), jnp.int32),
            pltpu.VMEM((_EMBED * 16 * 67,), jnp.float32),
            pltpu.VMEM((_TOK_PER_W // 128, 128), jnp.int32),
            pltpu.VMEM((_TOK_PER_W,), jnp.int32),
            pltpu.VMEM((2, _EMBED, _CHUNK), jnp.float32),
            pltpu.SemaphoreType.DMA,
        ],
    )
    wrep = jnp.pad(
        jnp.broadcast_to(weight.T[:, None, :], (_EMBED, 16, _NUM_EMB)),
        ((0, 0), (0, 0), (0, 67 - _NUM_EMB))).reshape(-1)
    out = run(query_residue_index,
              key_residue_index.reshape(_B * _L // 128, 128),
              wrep)
    return out.transpose(0, 2, 1)
